# baseline (speedup 1.0000x reference)
import functools

import jax
import jax.numpy as jnp
from jax import lax
from jax.experimental import pallas as pl
from jax.experimental.pallas import tpu as pltpu
from jax.experimental.pallas import tpu_sc as plsc

_FIELD_OFFSET = 1000000
_B = 4096
_D = 16
_ROWS = 2000000

_NC = 2
_NS = 16
_L = 16
_NW = _NC * _NS
_BPW = _B // _NW

_TILE_R = 128
_TILE_D = 8
_RT = _ROWS // _TILE_R
_DCONST = [(d // _TILE_D) * _RT * 1024 + (d % _TILE_D) * _TILE_R
           for d in range(_D)]


def _mf_body(x_hbm, t_hbm, out_hbm, xv, ub, ib, idxb, gath, outv, sem):
    wid = lax.axis_index("s") * _NC + lax.axis_index("c")

    pltpu.sync_copy(x_hbm.at[pl.ds(wid * 2 * _BPW, 2 * _BPW)], xv)

    for blk in range(_BPW // _L):
        sl = pl.ds(blk * _L, _L)
        u = xv[sl]
        it = xv[pl.ds(_BPW + blk * _L, _L)] + _FIELD_OFFSET
        ub[sl] = ((u >> 7) << 10) + (u & 127)
        ib[sl] = ((it >> 7) << 10) + (it & 127)

    copies = []
    for d in range(_D):
        for blk in range(_BPW // _L):
            sl = pl.ds(blk * _L, _L)
            idxb[d, sl] = ub[sl] + _DCONST[d]
        copies.append(pltpu.async_copy(t_hbm.at[idxb.at[d]], gath.at[d], sem))
    for d in range(_D):
        for blk in range(_BPW // _L):
            sl = pl.ds(blk * _L, _L)
            idxb[_D + d, sl] = ib[sl] + _DCONST[d]
        copies.append(
            pltpu.async_copy(t_hbm.at[idxb.at[_D + d]], gath.at[_D + d], sem)
        )
    for c in copies:
        c.wait()

    for blk in range(_BPW // _L):
        sl = pl.ds(blk * _L, _L)
        acc = gath[0, sl] * gath[_D, sl]
        for d in range(1, _D):
            acc = acc + gath[d, sl] * gath[_D + d, sl]
        outv[sl] = acc

    pltpu.sync_copy(outv, out_hbm.at[pl.ds(wid * _BPW, _BPW)])


@functools.partial(
    pl.kernel,
    out_type=jax.ShapeDtypeStruct((_B,), jnp.float32),
    mesh=plsc.VectorSubcoreMesh(core_axis_name="c", subcore_axis_name="s"),
    compiler_params=pltpu.CompilerParams(
        needs_layout_passes=False, use_tc_tiling_on_sc=False
    ),
    scratch_types=[
        pltpu.VMEM((2 * _BPW,), jnp.int32),
        pltpu.VMEM((_BPW,), jnp.int32),
        pltpu.VMEM((_BPW,), jnp.int32),
        pltpu.VMEM((2 * _D, _BPW), jnp.int32),
        pltpu.VMEM((2 * _D, _BPW), jnp.float32),
        pltpu.VMEM((_BPW,), jnp.float32),
        pltpu.SemaphoreType.DMA,
    ],
)
def _mf_kernel(x_hbm, t_hbm, out_hbm, xv, ub, ib, idxb, gath, outv, sem):
    _mf_body(x_hbm, t_hbm, out_hbm, xv, ub, ib, idxb, gath, outv, sem)


def kernel(x, table):
    xflat = (
        x.reshape(_B // _TILE_R, _TILE_R, 2)
        .transpose(0, 2, 1)
        .reshape(2 * _B)
    )
    tflat = (
        table.reshape(_RT, _TILE_R, _D // _TILE_D, _TILE_D)
        .transpose(2, 0, 3, 1)
        .reshape(_ROWS * _D)
    )
    y = _mf_kernel(xflat, tflat)
    return y.reshape(_B, 1)


# speedup vs baseline: 1.0093x; 1.0093x over previous
import functools

import jax
import jax.numpy as jnp
from jax import lax
from jax.experimental import pallas as pl
from jax.experimental.pallas import tpu as pltpu
from jax.experimental.pallas import tpu_sc as plsc

_FIELD_OFFSET = 1000000
_B = 4096
_D = 16
_ROWS = 2000000

_NC = 2
_NS = 16
_L = 16
_NW = _NC * _NS
_BPW = _B // _NW

_TILE_R = 128
_TILE_D = 8
_RT = _ROWS // _TILE_R
_DCONST = [(d // _TILE_D) * _RT * 1024 + (d % _TILE_D) * _TILE_R
           for d in range(_D)]


def _mf_body(x_hbm, t_hbm, out_hbm, xv, ub, ib, idxb, gath, outv, sem):
    wid = lax.axis_index("s") * _NC + lax.axis_index("c")

    pltpu.sync_copy(x_hbm.at[pl.ds(wid * 2 * _BPW, 2 * _BPW)], xv)

    for blk in range(_BPW // _L):
        sl = pl.ds(blk * _L, _L)
        u = xv[sl]
        it = xv[pl.ds(_BPW + blk * _L, _L)] + _FIELD_OFFSET
        ub[sl] = ((u >> 7) << 10) + (u & 127)
        ib[sl] = ((it >> 7) << 10) + (it & 127)

    for d in range(_D):
        for blk in range(_BPW // _L):
            sl = pl.ds(blk * _L, _L)
            idxb[pl.ds(d * _BPW + blk * _L, _L)] = ub[sl] + _DCONST[d]
        pltpu.async_copy(
            t_hbm.at[idxb.at[pl.ds(d * _BPW, _BPW)]],
            gath.at[pl.ds(d * _BPW, _BPW)], sem)
    for d in range(_D):
        for blk in range(_BPW // _L):
            sl = pl.ds(blk * _L, _L)
            idxb[pl.ds((_D + d) * _BPW + blk * _L, _L)] = ib[sl] + _DCONST[d]
        pltpu.async_copy(
            t_hbm.at[idxb.at[pl.ds((_D + d) * _BPW, _BPW)]],
            gath.at[pl.ds((_D + d) * _BPW, _BPW)], sem)
    pltpu.make_async_copy(t_hbm.at[pl.ds(0, 2 * _D * _BPW)], gath, sem).wait()

    for blk in range(_BPW // _L):
        sl = pl.ds(blk * _L, _L)
        j0 = blk * _L
        acc = gath[pl.ds(j0, _L)] * gath[pl.ds(_D * _BPW + j0, _L)]
        for d in range(1, _D):
            acc = acc + (gath[pl.ds(d * _BPW + j0, _L)]
                         * gath[pl.ds((_D + d) * _BPW + j0, _L)])
        outv[sl] = acc

    pltpu.sync_copy(outv, out_hbm.at[pl.ds(wid * _BPW, _BPW)])


@functools.partial(
    pl.kernel,
    out_type=jax.ShapeDtypeStruct((_B,), jnp.float32),
    mesh=plsc.VectorSubcoreMesh(core_axis_name="c", subcore_axis_name="s"),
    compiler_params=pltpu.CompilerParams(
        needs_layout_passes=False, use_tc_tiling_on_sc=False
    ),
    scratch_types=[
        pltpu.VMEM((2 * _BPW,), jnp.int32),
        pltpu.VMEM((_BPW,), jnp.int32),
        pltpu.VMEM((_BPW,), jnp.int32),
        pltpu.VMEM((2 * _D * _BPW,), jnp.int32),
        pltpu.VMEM((2 * _D * _BPW,), jnp.float32),
        pltpu.VMEM((_BPW,), jnp.float32),
        pltpu.SemaphoreType.DMA,
    ],
)
def _mf_kernel(x_hbm, t_hbm, out_hbm, xv, ub, ib, idxb, gath, outv, sem):
    _mf_body(x_hbm, t_hbm, out_hbm, xv, ub, ib, idxb, gath, outv, sem)


def kernel(x, table):
    xflat = (
        x.reshape(_B // _TILE_R, _TILE_R, 2)
        .transpose(0, 2, 1)
        .reshape(2 * _B)
    )
    tflat = (
        table.reshape(_RT, _TILE_R, _D // _TILE_D, _TILE_D)
        .transpose(2, 0, 3, 1)
        .reshape(_ROWS * _D)
    )
    y = _mf_kernel(xflat, tflat)
    return y.reshape(_B, 1)


# speedup vs baseline: 1.0253x; 1.0158x over previous
import functools

import jax
import jax.numpy as jnp
from jax import lax
from jax.experimental import pallas as pl
from jax.experimental.pallas import tpu as pltpu
from jax.experimental.pallas import tpu_sc as plsc

_FIELD_OFFSET = 1000000
_B = 4096
_D = 16
_ROWS = 2000000

_NC = 2
_NS = 16
_L = 16
_NW = _NC * _NS
_BPW = _B // _NW

_TILE_R = 128
_TILE_D = 8
_RT = _ROWS // _TILE_R
_DCONST = [(d // _TILE_D) * _RT * 1024 + (d % _TILE_D) * _TILE_R
           for d in range(_D)]


def _mf_body(x_hbm, t_hbm, out_hbm, xv, ub, ib, idxb, gath, outv, sem):
    wid = lax.axis_index("s") * _NC + lax.axis_index("c")

    pltpu.sync_copy(x_hbm.at[pl.ds(wid * 2 * _BPW, 2 * _BPW)], xv)

    for blk in range(_BPW // _L):
        sl = pl.ds(blk * _L, _L)
        u = xv[sl]
        it = xv[pl.ds(_BPW + blk * _L, _L)] + _FIELD_OFFSET
        ub[sl] = ((u >> 7) << 10) + (u & 127)
        ib[sl] = ((it >> 7) << 10) + (it & 127)

    for d in range(_D):
        for blk in range(_BPW // _L):
            sl = pl.ds(blk * _L, _L)
            idxb[pl.ds(d * _BPW + blk * _L, _L)] = ub[sl] + _DCONST[d]
    for d in range(_D):
        for blk in range(_BPW // _L):
            sl = pl.ds(blk * _L, _L)
            idxb[pl.ds((_D + d) * _BPW + blk * _L, _L)] = ib[sl] + _DCONST[d]
    pltpu.async_copy(t_hbm.at[idxb], gath, sem).wait()

    for blk in range(_BPW // _L):
        sl = pl.ds(blk * _L, _L)
        j0 = blk * _L
        acc = gath[pl.ds(j0, _L)] * gath[pl.ds(_D * _BPW + j0, _L)]
        for d in range(1, _D):
            acc = acc + (gath[pl.ds(d * _BPW + j0, _L)]
                         * gath[pl.ds((_D + d) * _BPW + j0, _L)])
        outv[sl] = acc

    pltpu.sync_copy(outv, out_hbm.at[pl.ds(wid * _BPW, _BPW)])


@functools.partial(
    pl.kernel,
    out_type=jax.ShapeDtypeStruct((_B,), jnp.float32),
    mesh=plsc.VectorSubcoreMesh(core_axis_name="c", subcore_axis_name="s"),
    compiler_params=pltpu.CompilerParams(
        needs_layout_passes=False, use_tc_tiling_on_sc=False
    ),
    scratch_types=[
        pltpu.VMEM((2 * _BPW,), jnp.int32),
        pltpu.VMEM((_BPW,), jnp.int32),
        pltpu.VMEM((_BPW,), jnp.int32),
        pltpu.VMEM((2 * _D * _BPW,), jnp.int32),
        pltpu.VMEM((2 * _D * _BPW,), jnp.float32),
        pltpu.VMEM((_BPW,), jnp.float32),
        pltpu.SemaphoreType.DMA,
    ],
)
def _mf_kernel(x_hbm, t_hbm, out_hbm, xv, ub, ib, idxb, gath, outv, sem):
    _mf_body(x_hbm, t_hbm, out_hbm, xv, ub, ib, idxb, gath, outv, sem)


def kernel(x, table):
    xflat = (
        x.reshape(_B // _TILE_R, _TILE_R, 2)
        .transpose(0, 2, 1)
        .reshape(2 * _B)
    )
    tflat = (
        table.reshape(_RT, _TILE_R, _D // _TILE_D, _TILE_D)
        .transpose(2, 0, 3, 1)
        .reshape(_ROWS * _D)
    )
    y = _mf_kernel(xflat, tflat)
    return y.reshape(_B, 1)


# speedup vs baseline: 1.0399x; 1.0143x over previous
import functools

import jax
import jax.numpy as jnp
from jax import lax
from jax.experimental import pallas as pl
from jax.experimental.pallas import tpu as pltpu
from jax.experimental.pallas import tpu_sc as plsc

_FIELD_OFFSET = 1000000
_B = 4096
_D = 16
_ROWS = 2000000

_NC = 2
_NS = 16
_L = 16
_NW = _NC * _NS
_BPW = _B // _NW

_TILE_R = 128
_TILE_D = 8
_RT = _ROWS // _TILE_R
_DCONST = [(d // _TILE_D) * _RT * 1024 + (d % _TILE_D) * _TILE_R
           for d in range(_D)]


def _mf_body(x_hbm, t_hbm, out_hbm, xv, idxb, gath, outv, sem):
    wid = lax.axis_index("s") * _NC + lax.axis_index("c")

    pltpu.sync_copy(x_hbm.at[pl.ds(wid * 2 * _BPW, 2 * _BPW)], xv)

    for blk in range(_BPW // _L):
        sl = pl.ds(blk * _L, _L)
        u = xv[sl]
        ubase = ((u >> 7) << 10) + (u & 127)
        for d in range(_D):
            idxb[pl.ds(d * _BPW + blk * _L, _L)] = ubase + _DCONST[d]
    cu = pltpu.async_copy(
        t_hbm.at[idxb.at[pl.ds(0, _D * _BPW)]],
        gath.at[pl.ds(0, _D * _BPW)], sem)
    for blk in range(_BPW // _L):
        it = xv[pl.ds(_BPW + blk * _L, _L)] + _FIELD_OFFSET
        ibase = ((it >> 7) << 10) + (it & 127)
        for d in range(_D):
            idxb[pl.ds((_D + d) * _BPW + blk * _L, _L)] = ibase + _DCONST[d]
    ci = pltpu.async_copy(
        t_hbm.at[idxb.at[pl.ds(_D * _BPW, _D * _BPW)]],
        gath.at[pl.ds(_D * _BPW, _D * _BPW)], sem)
    cu.wait()
    ci.wait()

    for blk in range(_BPW // _L):
        sl = pl.ds(blk * _L, _L)
        j0 = blk * _L
        acc = gath[pl.ds(j0, _L)] * gath[pl.ds(_D * _BPW + j0, _L)]
        for d in range(1, _D):
            acc = acc + (gath[pl.ds(d * _BPW + j0, _L)]
                         * gath[pl.ds((_D + d) * _BPW + j0, _L)])
        outv[sl] = acc

    pltpu.sync_copy(outv, out_hbm.at[pl.ds(wid * _BPW, _BPW)])


@functools.partial(
    pl.kernel,
    out_type=jax.ShapeDtypeStruct((_B,), jnp.float32),
    mesh=plsc.VectorSubcoreMesh(core_axis_name="c", subcore_axis_name="s"),
    compiler_params=pltpu.CompilerParams(
        needs_layout_passes=False, use_tc_tiling_on_sc=False
    ),
    scratch_types=[
        pltpu.VMEM((2 * _BPW,), jnp.int32),
        pltpu.VMEM((2 * _D * _BPW,), jnp.int32),
        pltpu.VMEM((2 * _D * _BPW,), jnp.float32),
        pltpu.VMEM((_BPW,), jnp.float32),
        pltpu.SemaphoreType.DMA,
    ],
)
def _mf_kernel(x_hbm, t_hbm, out_hbm, xv, idxb, gath, outv, sem):
    _mf_body(x_hbm, t_hbm, out_hbm, xv, idxb, gath, outv, sem)


def kernel(x, table):
    xflat = (
        x.reshape(_B // _TILE_R, _TILE_R, 2)
        .transpose(0, 2, 1)
        .reshape(2 * _B)
    )
    tflat = (
        table.reshape(_RT, _TILE_R, _D // _TILE_D, _TILE_D)
        .transpose(2, 0, 3, 1)
        .reshape(_ROWS * _D)
    )
    y = _mf_kernel(xflat, tflat)
    return y.reshape(_B, 1)


# speedup vs baseline: 1.0757x; 1.0345x over previous
import functools

import jax
import jax.numpy as jnp
from jax import lax
from jax.experimental import pallas as pl
from jax.experimental.pallas import tpu as pltpu
from jax.experimental.pallas import tpu_sc as plsc

_FIELD_OFFSET = 1000000
_B = 4096
_D = 16
_ROWS = 2000000

_NC = 2
_NS = 16
_L = 16
_NW = _NC * _NS
_BPW = _B // _NW

_TILE_R = 128
_TILE_D = 8
_RT = _ROWS // _TILE_R
_DCONST = [(d // _TILE_D) * _RT * 1024 + (d % _TILE_D) * _TILE_R
           for d in range(_D)]


def _mf_body(x_hbm, t_hbm, out_hbm, xv, idxb, gath, outv, sem):
    wid = lax.axis_index("s") * _NC + lax.axis_index("c")

    pltpu.sync_copy(x_hbm.at[pl.ds(wid * 2 * _BPW, 2 * _BPW)], xv)

    def _build(blk, base_off):
        j0 = blk * _L
        v = xv[pl.ds(base_off * _BPW + j0, _L)]
        v = jnp.where(base_off == 1, v + _FIELD_OFFSET, v)
        vb = ((v >> 7) << 10) + (v & 127)
        for d in range(_D):
            idxb[pl.ds((base_off * _D + d) * _BPW + j0, _L)] = vb + _DCONST[d]
        return 0

    lax.fori_loop(0, _BPW // _L, lambda blk, c: _build(blk, 0), 0,
                  unroll=False)
    cu = pltpu.async_copy(
        t_hbm.at[idxb.at[pl.ds(0, _D * _BPW)]],
        gath.at[pl.ds(0, _D * _BPW)], sem)
    lax.fori_loop(0, _BPW // _L, lambda blk, c: _build(blk, 1), 0,
                  unroll=False)
    ci = pltpu.async_copy(
        t_hbm.at[idxb.at[pl.ds(_D * _BPW, _D * _BPW)]],
        gath.at[pl.ds(_D * _BPW, _D * _BPW)], sem)
    cu.wait()
    ci.wait()

    def _dot(blk, c):
        j0 = blk * _L
        acc = gath[pl.ds(j0, _L)] * gath[pl.ds(_D * _BPW + j0, _L)]
        for d in range(1, _D):
            acc = acc + (gath[pl.ds(d * _BPW + j0, _L)]
                         * gath[pl.ds((_D + d) * _BPW + j0, _L)])
        outv[pl.ds(j0, _L)] = acc
        return 0

    lax.fori_loop(0, _BPW // _L, _dot, 0, unroll=False)

    pltpu.sync_copy(outv, out_hbm.at[pl.ds(wid * _BPW, _BPW)])


@functools.partial(
    pl.kernel,
    out_type=jax.ShapeDtypeStruct((_B,), jnp.float32),
    mesh=plsc.VectorSubcoreMesh(core_axis_name="c", subcore_axis_name="s"),
    compiler_params=pltpu.CompilerParams(
        needs_layout_passes=False, use_tc_tiling_on_sc=False
    ),
    scratch_types=[
        pltpu.VMEM((2 * _BPW,), jnp.int32),
        pltpu.VMEM((2 * _D * _BPW,), jnp.int32),
        pltpu.VMEM((2 * _D * _BPW,), jnp.float32),
        pltpu.VMEM((_BPW,), jnp.float32),
        pltpu.SemaphoreType.DMA,
    ],
)
def _mf_kernel(x_hbm, t_hbm, out_hbm, xv, idxb, gath, outv, sem):
    _mf_body(x_hbm, t_hbm, out_hbm, xv, idxb, gath, outv, sem)


def kernel(x, table):
    xflat = (
        x.reshape(_B // _TILE_R, _TILE_R, 2)
        .transpose(0, 2, 1)
        .reshape(2 * _B)
    )
    tflat = (
        table.reshape(_RT, _TILE_R, _D // _TILE_D, _TILE_D)
        .transpose(2, 0, 3, 1)
        .reshape(_ROWS * _D)
    )
    y = _mf_kernel(xflat, tflat)
    return y.reshape(_B, 1)


# speedup vs baseline: 1.0766x; 1.0008x over previous
import functools

import jax
import jax.numpy as jnp
from jax import lax
from jax.experimental import pallas as pl
from jax.experimental.pallas import tpu as pltpu
from jax.experimental.pallas import tpu_sc as plsc

_FIELD_OFFSET = 1000000
_B = 4096
_D = 16
_ROWS = 2000000

_NC = 2
_NS = 16
_L = 16
_NW = _NC * _NS
_BPW = _B // _NW

_TILE_R = 128
_TILE_D = 8
_RT = _ROWS // _TILE_R
_DCONST = [(d // _TILE_D) * _RT * 1024 + (d % _TILE_D) * _TILE_R
           for d in range(_D)]


def _mf_body(x_hbm, t_hbm, out_hbm, xv, idxb, gath, outv, sem, sem_x):
    wid = lax.axis_index("s") * _NC + lax.axis_index("c")

    ci_x = pltpu.async_copy(
        x_hbm.at[pl.ds(wid * 2 * _BPW + _BPW, _BPW)],
        xv.at[pl.ds(_BPW, _BPW)], sem_x)
    pltpu.sync_copy(x_hbm.at[pl.ds(wid * 2 * _BPW, _BPW)],
                    xv.at[pl.ds(0, _BPW)])

    def _build(blk, base_off):
        j0 = blk * _L
        v = xv[pl.ds(base_off * _BPW + j0, _L)]
        v = jnp.where(base_off == 1, v + _FIELD_OFFSET, v)
        vb = ((v >> 7) << 10) + (v & 127)
        for d in range(_D):
            idxb[pl.ds((base_off * _D + d) * _BPW + j0, _L)] = vb + _DCONST[d]
        return 0

    lax.fori_loop(0, _BPW // _L, lambda blk, c: _build(blk, 0), 0,
                  unroll=False)
    cu = pltpu.async_copy(
        t_hbm.at[idxb.at[pl.ds(0, _D * _BPW)]],
        gath.at[pl.ds(0, _D * _BPW)], sem)
    ci_x.wait()
    lax.fori_loop(0, _BPW // _L, lambda blk, c: _build(blk, 1), 0,
                  unroll=False)
    ci = pltpu.async_copy(
        t_hbm.at[idxb.at[pl.ds(_D * _BPW, _D * _BPW)]],
        gath.at[pl.ds(_D * _BPW, _D * _BPW)], sem)
    cu.wait()
    ci.wait()

    def _dot(blk, c):
        j0 = blk * _L
        acc = gath[pl.ds(j0, _L)] * gath[pl.ds(_D * _BPW + j0, _L)]
        for d in range(1, _D):
            acc = acc + (gath[pl.ds(d * _BPW + j0, _L)]
                         * gath[pl.ds((_D + d) * _BPW + j0, _L)])
        outv[pl.ds(j0, _L)] = acc
        return 0

    lax.fori_loop(0, _BPW // _L, _dot, 0, unroll=False)

    pltpu.sync_copy(outv, out_hbm.at[pl.ds(wid * _BPW, _BPW)])


@functools.partial(
    pl.kernel,
    out_type=jax.ShapeDtypeStruct((_B,), jnp.float32),
    mesh=plsc.VectorSubcoreMesh(core_axis_name="c", subcore_axis_name="s"),
    compiler_params=pltpu.CompilerParams(
        needs_layout_passes=False, use_tc_tiling_on_sc=False
    ),
    scratch_types=[
        pltpu.VMEM((2 * _BPW,), jnp.int32),
        pltpu.VMEM((2 * _D * _BPW,), jnp.int32),
        pltpu.VMEM((2 * _D * _BPW,), jnp.float32),
        pltpu.VMEM((_BPW,), jnp.float32),
        pltpu.SemaphoreType.DMA,
        pltpu.SemaphoreType.DMA,
    ],
)
def _mf_kernel(x_hbm, t_hbm, out_hbm, xv, idxb, gath, outv, sem, sem_x):
    _mf_body(x_hbm, t_hbm, out_hbm, xv, idxb, gath, outv, sem, sem_x)


def kernel(x, table):
    xflat = (
        x.reshape(_B // _TILE_R, _TILE_R, 2)
        .transpose(0, 2, 1)
        .reshape(2 * _B)
    )
    tflat = (
        table.reshape(_RT, _TILE_R, _D // _TILE_D, _TILE_D)
        .transpose(2, 0, 3, 1)
        .reshape(_ROWS * _D)
    )
    y = _mf_kernel(xflat, tflat)
    return y.reshape(_B, 1)
